# Initial kernel scaffold; baseline (speedup 1.0000x reference)
import jax
import jax.numpy as jnp
from jax.experimental import pallas as pl


def kernel(embeddings, click_labels, cas_supp, label, cls_num):
    raise NotImplementedError("write your pallas kernel here")


# speedup vs baseline: 1.0059x; 1.0059x over previous
import functools

import jax
import jax.numpy as jnp
from jax import lax
from jax.experimental import pallas as pl
from jax.experimental.pallas import tpu as pltpu

K = 64
TH_SIMILAR_MIN = 0.5
TH_DIFFERENT_MAX = 0.1
_PREC = lax.Precision.HIGHEST


def _rowsum(rowvec, colvec):
    return lax.dot_general(
        rowvec, colvec, (((1,), (0,)), ((), ())),
        preferred_element_type=jnp.float32, precision=_PREC)[0, 0]


def _body(cls_ref, emb_ref, click_ref, cas_ref, lab_ref,
          loss_ref, val_ref, nref, mref, *, rt):
    d, t = nref.shape
    c = cas_ref.shape[1]
    nt = t // rt

    e = emb_ref[0]
    n2 = jnp.sum(e * e, axis=0, keepdims=True)
    inv = 1.0 / jnp.sqrt(jnp.maximum(n2, 1e-24))
    nref[...] = e * inv

    cls = cls_ref[0, 0]
    click = click_ref[0]
    bgf = (click == cls).astype(jnp.float32)
    c_bg = jnp.sum(bgf)

    cas = cas_ref[0]
    col = lax.broadcasted_iota(jnp.int32, (c, t), 1)

    def step(_, carry):
        cur, sel = carry
        m = jnp.max(cur, axis=1, keepdims=True)
        ismax = cur == m
        first = jnp.min(jnp.where(ismax, col, t), axis=1, keepdims=True)
        pick = col == first
        sel = jnp.where(pick, 1.0, sel)
        cur = jnp.where(pick, -jnp.inf, cur)
        return cur, sel

    _, sel = lax.fori_loop(0, K, step,
                           (cas, jnp.zeros((c, t), jnp.float32)))

    labf = (lab_ref[0] == 1).astype(jnp.float32)
    w = lax.dot_general(labf, sel, (((1,), (0,)), ((), ())),
                        preferred_element_type=jnp.float32,
                        precision=_PREC)
    c_fg = jnp.sum(w)
    fgp = jnp.where(w > 0, 1.0, 0.0)

    mref[0:1, :] = bgf
    mref[1:2, :] = w
    mref[2:3, :] = fgp

    s1 = jnp.float32(0.0)
    s2 = jnp.float32(0.0)
    s3 = jnp.float32(0.0)
    s4 = jnp.float32(0.0)
    for r in range(nt):
        nr = nref[:, r * rt:(r + 1) * rt]
        amin = jnp.full((rt, 1), jnp.inf, jnp.float32)
        bmin = jnp.full((rt, 1), jnp.inf, jnp.float32)
        cmax = jnp.full((rt, 1), -jnp.inf, jnp.float32)
        dmax = jnp.full((rt, 1), -jnp.inf, jnp.float32)
        for cc in range(nt):
            nc = nref[:, cc * rt:(cc + 1) * rt]
            g = lax.dot_general(nr, nc, (((0,), (0,)), ((), ())),
                                preferred_element_type=jnp.float32,
                                precision=_PREC)
            fg_c = mref[2:3, cc * rt:(cc + 1) * rt] > 0
            bg_c = mref[0:1, cc * rt:(cc + 1) * rt] > 0
            amin = jnp.minimum(amin, jnp.min(
                jnp.where(fg_c, g, jnp.inf), axis=1, keepdims=True))
            bmin = jnp.minimum(bmin, jnp.min(
                jnp.where(bg_c, g, jnp.inf), axis=1, keepdims=True))
            cmax = jnp.maximum(cmax, jnp.max(
                jnp.where(bg_c, g, -jnp.inf), axis=1, keepdims=True))
            dmax = jnp.maximum(dmax, jnp.max(
                jnp.where(fg_c, g, -jnp.inf), axis=1, keepdims=True))
        w_r = mref[1:2, r * rt:(r + 1) * rt]
        bg_r = mref[0:1, r * rt:(r + 1) * rt]
        s1 = s1 + _rowsum(w_r, jnp.maximum(TH_SIMILAR_MIN - amin, 0.0))
        s2 = s2 + _rowsum(bg_r, jnp.maximum(TH_SIMILAR_MIN - bmin, 0.0))
        s3 = s3 + _rowsum(w_r, jnp.maximum(cmax - TH_DIFFERENT_MAX, 0.0))
        s4 = s4 + _rowsum(bg_r, jnp.maximum(dmax - TH_DIFFERENT_MAX, 0.0))

    loss_b = (s1 + s3) / c_fg + (s2 + s4) / c_bg
    valid = c_bg > 0
    loss_ref[...] = jnp.zeros((1, 8, 128), jnp.float32) + jnp.where(
        valid, loss_b, 0.0)
    val_ref[...] = jnp.zeros((1, 8, 128), jnp.float32) + jnp.where(
        valid, 1.0, 0.0)


def kernel(embeddings, click_labels, cas_supp, label, cls_num):
    b, d, t = embeddings.shape
    c = cas_supp.shape[1]
    rt = min(512, t)

    cls_arr = jnp.asarray(cls_num, jnp.int32).reshape(1, 1)
    click3 = click_labels.reshape(b, 1, t)
    lab3 = label.reshape(b, 1, c)

    lossm, validf = pl.pallas_call(
        functools.partial(_body, rt=rt),
        grid=(b,),
        in_specs=[
            pl.BlockSpec((1, 1), lambda i: (0, 0),
                         memory_space=pltpu.SMEM),
            pl.BlockSpec((1, d, t), lambda i: (i, 0, 0)),
            pl.BlockSpec((1, 1, t), lambda i: (i, 0, 0)),
            pl.BlockSpec((1, c, t), lambda i: (i, 0, 0)),
            pl.BlockSpec((1, 1, c), lambda i: (i, 0, 0)),
        ],
        out_specs=[
            pl.BlockSpec((1, 8, 128), lambda i: (i, 0, 0)),
            pl.BlockSpec((1, 8, 128), lambda i: (i, 0, 0)),
        ],
        out_shape=[
            jax.ShapeDtypeStruct((b, 8, 128), jnp.float32),
            jax.ShapeDtypeStruct((b, 8, 128), jnp.float32),
        ],
        scratch_shapes=[
            pltpu.VMEM((d, t), jnp.float32),
            pltpu.VMEM((8, t), jnp.float32),
        ],
    )(cls_arr, embeddings, click3, cas_supp, lab3)

    return jnp.sum(lossm[:, 0, 0]) / jnp.sum(validf[:, 0, 0])


# speedup vs baseline: 1.7886x; 1.7781x over previous
import functools

import jax
import jax.numpy as jnp
from jax import lax
from jax.experimental import pallas as pl
from jax.experimental.pallas import tpu as pltpu

K = 64
TH_SIMILAR_MIN = 0.5
TH_DIFFERENT_MAX = 0.1
_PREC = lax.Precision.DEFAULT


def _rowsum(rowvec, colvec):
    return lax.dot_general(
        rowvec, colvec, (((1,), (0,)), ((), ())),
        preferred_element_type=jnp.float32, precision=_PREC)[0, 0]


def _body(cls_ref, emb_ref, click_ref, cas_ref, lab_ref,
          loss_ref, val_ref, nref, mref, *, rt):
    d, t = nref.shape
    c = cas_ref.shape[1]
    nt = t // rt

    e = emb_ref[0]
    n2 = jnp.sum(e * e, axis=0, keepdims=True)
    inv = 1.0 / jnp.sqrt(jnp.maximum(n2, 1e-24))
    nref[...] = e * inv

    cls = cls_ref[0, 0]
    click = click_ref[0]
    bgf = (click == cls).astype(jnp.float32)
    c_bg = jnp.sum(bgf)

    cas = cas_ref[0]
    col = lax.broadcasted_iota(jnp.int32, (c, t), 1)

    def step(_, carry):
        cur, sel = carry
        m = jnp.max(cur, axis=1, keepdims=True)
        ismax = cur == m
        first = jnp.min(jnp.where(ismax, col, t), axis=1, keepdims=True)
        pick = col == first
        sel = jnp.where(pick, 1.0, sel)
        cur = jnp.where(pick, -jnp.inf, cur)
        return cur, sel

    _, sel = lax.fori_loop(0, K, step,
                           (cas, jnp.zeros((c, t), jnp.float32)))

    labf = (lab_ref[0] == 1).astype(jnp.float32)
    w = lax.dot_general(labf, sel, (((1,), (0,)), ((), ())),
                        preferred_element_type=jnp.float32,
                        precision=_PREC)
    c_fg = jnp.sum(w)
    fgp = jnp.where(w > 0, 1.0, 0.0)

    mref[0:1, :] = bgf
    mref[1:2, :] = w
    mref[2:3, :] = fgp

    s1 = jnp.float32(0.0)
    s2 = jnp.float32(0.0)
    s3 = jnp.float32(0.0)
    s4 = jnp.float32(0.0)
    for r in range(nt):
        nr = nref[:, r * rt:(r + 1) * rt]
        amin = jnp.full((rt, 1), jnp.inf, jnp.float32)
        bmin = jnp.full((rt, 1), jnp.inf, jnp.float32)
        cmax = jnp.full((rt, 1), -jnp.inf, jnp.float32)
        dmax = jnp.full((rt, 1), -jnp.inf, jnp.float32)
        for cc in range(nt):
            nc = nref[:, cc * rt:(cc + 1) * rt]
            g = lax.dot_general(nr, nc, (((0,), (0,)), ((), ())),
                                preferred_element_type=jnp.float32,
                                precision=_PREC)
            fg_c = mref[2:3, cc * rt:(cc + 1) * rt] > 0
            bg_c = mref[0:1, cc * rt:(cc + 1) * rt] > 0
            amin = jnp.minimum(amin, jnp.min(
                jnp.where(fg_c, g, jnp.inf), axis=1, keepdims=True))
            bmin = jnp.minimum(bmin, jnp.min(
                jnp.where(bg_c, g, jnp.inf), axis=1, keepdims=True))
            cmax = jnp.maximum(cmax, jnp.max(
                jnp.where(bg_c, g, -jnp.inf), axis=1, keepdims=True))
            dmax = jnp.maximum(dmax, jnp.max(
                jnp.where(fg_c, g, -jnp.inf), axis=1, keepdims=True))
        w_r = mref[1:2, r * rt:(r + 1) * rt]
        bg_r = mref[0:1, r * rt:(r + 1) * rt]
        s1 = s1 + _rowsum(w_r, jnp.maximum(TH_SIMILAR_MIN - amin, 0.0))
        s2 = s2 + _rowsum(bg_r, jnp.maximum(TH_SIMILAR_MIN - bmin, 0.0))
        s3 = s3 + _rowsum(w_r, jnp.maximum(cmax - TH_DIFFERENT_MAX, 0.0))
        s4 = s4 + _rowsum(bg_r, jnp.maximum(dmax - TH_DIFFERENT_MAX, 0.0))

    loss_b = (s1 + s3) / c_fg + (s2 + s4) / c_bg
    valid = c_bg > 0
    loss_ref[...] = jnp.zeros((1, 8, 128), jnp.float32) + jnp.where(
        valid, loss_b, 0.0)
    val_ref[...] = jnp.zeros((1, 8, 128), jnp.float32) + jnp.where(
        valid, 1.0, 0.0)


def kernel(embeddings, click_labels, cas_supp, label, cls_num):
    b, d, t = embeddings.shape
    c = cas_supp.shape[1]
    rt = min(512, t)

    cls_arr = jnp.asarray(cls_num, jnp.int32).reshape(1, 1)
    click3 = click_labels.reshape(b, 1, t)
    lab3 = label.reshape(b, 1, c)

    lossm, validf = pl.pallas_call(
        functools.partial(_body, rt=rt),
        grid=(b,),
        in_specs=[
            pl.BlockSpec((1, 1), lambda i: (0, 0),
                         memory_space=pltpu.SMEM),
            pl.BlockSpec((1, d, t), lambda i: (i, 0, 0)),
            pl.BlockSpec((1, 1, t), lambda i: (i, 0, 0)),
            pl.BlockSpec((1, c, t), lambda i: (i, 0, 0)),
            pl.BlockSpec((1, 1, c), lambda i: (i, 0, 0)),
        ],
        out_specs=[
            pl.BlockSpec((1, 8, 128), lambda i: (i, 0, 0)),
            pl.BlockSpec((1, 8, 128), lambda i: (i, 0, 0)),
        ],
        out_shape=[
            jax.ShapeDtypeStruct((b, 8, 128), jnp.float32),
            jax.ShapeDtypeStruct((b, 8, 128), jnp.float32),
        ],
        scratch_shapes=[
            pltpu.VMEM((d, t), jnp.float32),
            pltpu.VMEM((8, t), jnp.float32),
        ],
    )(cls_arr, embeddings, click3, cas_supp, lab3)

    return jnp.sum(lossm[:, 0, 0]) / jnp.sum(validf[:, 0, 0])


# speedup vs baseline: 2.5696x; 1.4367x over previous
import functools

import jax
import jax.numpy as jnp
from jax import lax
from jax.experimental import pallas as pl
from jax.experimental.pallas import tpu as pltpu
from jax.experimental.pallas import tpu_sc as plsc

K = 64
TH_SIMILAR_MIN = 0.5
TH_DIFFERENT_MAX = 0.1
_PREC = lax.Precision.DEFAULT
FT = 256


def _topk_body(cas_ref, sel_ref):
    c, t = cas_ref.shape[1], cas_ref.shape[2]
    cas = cas_ref[0]
    col = lax.broadcasted_iota(jnp.int32, (c, t), 1)

    def step(_, carry):
        cur, sel = carry
        m = jnp.max(cur, axis=1, keepdims=True)
        ismax = cur == m
        first = jnp.min(jnp.where(ismax, col, t), axis=1, keepdims=True)
        pick = col == first
        sel = jnp.where(pick, 1.0, sel)
        cur = jnp.where(pick, -jnp.inf, cur)
        return cur, sel

    _, sel = lax.fori_loop(0, K, step,
                           (cas, jnp.zeros((c, t), jnp.float32)))
    sel_ref[0] = sel


def _topk_sel(cas_supp):
    b, c, t = cas_supp.shape
    return pl.pallas_call(
        _topk_body,
        grid=(b,),
        in_specs=[pl.BlockSpec((1, c, t), lambda i: (i, 0, 0))],
        out_specs=pl.BlockSpec((1, c, t), lambda i: (i, 0, 0)),
        out_shape=jax.ShapeDtypeStruct((b, c, t), jnp.float32),
    )(cas_supp)


def _compact(src_ref, idx_ref, match_fn, t):
    def body(i, cursor):
        v = src_ref[pl.ds(i * 16, 16)]
        m = match_fn(v)
        mi = jnp.where(m, jnp.full((16,), 1, jnp.int32),
                       jnp.zeros((16,), jnp.int32))
        excl = plsc.cumsum(mi) - mi
        idx = lax.iota(jnp.int32, 16) + i * 16
        plsc.store_scatter(idx_ref, [excl + cursor], idx, mask=m)
        return cursor + jnp.sum(mi)
    return lax.fori_loop(0, t // 16, body, jnp.int32(0))


def _sc_gather(sel, click, etr, cls_arr):
    b, c, t = sel.shape
    d = etr.shape[2]
    nfg = c * K

    mesh = plsc.VectorSubcoreMesh(core_axis_name="c", subcore_axis_name="s")

    @functools.partial(
        pl.kernel, mesh=mesh,
        compiler_params=pltpu.CompilerParams(needs_layout_passes=False),
        out_type=[
            jax.ShapeDtypeStruct((b, nfg, d), jnp.float32),
            jax.ShapeDtypeStruct((b, t, d), jnp.float32),
            jax.ShapeDtypeStruct((b, 16), jnp.int32),
        ],
        scratch_types=[
            pltpu.VMEM((t,), jnp.float32),
            pltpu.VMEM((t,), jnp.int32),
            pltpu.VMEM((t + 32,), jnp.int32),
            pltpu.VMEM((K,), jnp.int32),
            pltpu.VMEM((128,), jnp.int32),
            pltpu.VMEM((K, 128), jnp.float32),
            pltpu.VMEM((128, 128), jnp.float32),
            pltpu.VMEM((16,), jnp.int32),
            pltpu.SemaphoreType.DMA,
        ],
    )
    def sck(sel_h, click_h, etr_h, cls_h,
            fg_h, bg_h, cnt_h,
            selv, clickv, idxv, idxfg, idxch, rowsf, rows, sv, sem):
        wid = lax.axis_index("s") * 2 + lax.axis_index("c")

        for i in range(10):
            task = wid * 10 + i
            bb = task // c
            cc = task - bb * c
            pltpu.sync_copy(sel_h.at[bb].at[cc], selv)
            n = _compact(selv, idxfg, lambda v: v > 0.5, t)
            del n
            pltpu.async_copy(etr_h.at[bb].at[idxfg], rowsf, sem).wait()
            pltpu.sync_copy(rowsf, fg_h.at[bb].at[pl.ds(cc * K, K)])

        @pl.when(wid < b)
        def _():
            bb = wid
            pltpu.sync_copy(cls_h, sv)
            clsv = sv[...]
            pltpu.sync_copy(click_h.at[bb], clickv)

            def zero(i, _):
                idxv[pl.ds(i * 16, 16)] = jnp.zeros((16,), jnp.int32)
                return 0
            lax.fori_loop(0, (t + 32) // 16, zero, 0)

            def body(i, cursor):
                v = clickv[pl.ds(i * 16, 16)]
                m = v == clsv
                mi = jnp.where(m, jnp.full((16,), 1, jnp.int32),
                               jnp.zeros((16,), jnp.int32))
                excl = plsc.cumsum(mi) - mi
                idx = lax.iota(jnp.int32, 16) + i * 16
                plsc.store_scatter(idxv, [excl + cursor], idx, mask=m)
                return cursor + jnp.sum(mi)
            cnt = lax.fori_loop(0, t // 16, body, jnp.int32(0))

            sv[...] = jnp.zeros((16,), jnp.int32) + cnt
            pltpu.sync_copy(sv, cnt_h.at[bb])

            nch = (cnt + 127) // 128

            def gat(j, _):
                for k in range(8):
                    idxch[pl.ds(k * 16, 16)] = idxv[
                        pl.ds(j * 128 + k * 16, 16)]
                pltpu.async_copy(etr_h.at[bb].at[idxch], rows, sem).wait()
                pltpu.sync_copy(rows, bg_h.at[bb].at[pl.ds(j * 128, 128)])
                return 0
            lax.fori_loop(0, nch, gat, 0)

    return sck(sel, click, etr, cls_arr)


def _dotT(a, bmat):
    return lax.dot_general(a, bmat, (((1,), (1,)), ((), ())),
                           preferred_element_type=jnp.float32,
                           precision=_PREC)


def _rowsum(rowvec, colvec):
    return lax.dot_general(rowvec, colvec, (((1,), (0,)), ((), ())),
                           preferred_element_type=jnp.float32,
                           precision=_PREC)[0, 0]


def _main_body(cnt_ref, fg_ref, lab_ref, bg_ref, loss_ref, val_ref,
               fn_ref, bn_ref):
    nfg, d = fn_ref.shape
    t = bn_ref.shape[0]
    c = lab_ref.shape[2]
    nft = nfg // FT

    cnt = cnt_ref[0, 0, 0]
    c_bg = cnt.astype(jnp.float32)
    labf = (lab_ref[0] == 1).astype(jnp.float32)
    rsel = (lax.broadcasted_iota(jnp.int32, (c, nfg), 1) // K
            == lax.broadcasted_iota(jnp.int32, (c, nfg), 0)
            ).astype(jnp.float32)
    wgt = lax.dot_general(labf, rsel, (((1,), (0,)), ((), ())),
                          preferred_element_type=jnp.float32,
                          precision=_PREC)
    c_fg = jnp.sum(wgt)
    nbt = lax.div(cnt + (FT - 1), FT)

    for i in range(nft):
        rows = fg_ref[0, i * FT:(i + 1) * FT, :]
        n2 = jnp.sum(rows * rows, axis=1, keepdims=True)
        fn_ref[i * FT:(i + 1) * FT, :] = rows / jnp.sqrt(
            jnp.maximum(n2, 1e-24))

    def nrm(j, _):
        rows = bg_ref[0, pl.ds(j * FT, FT), :]
        n2 = jnp.sum(rows * rows, axis=1, keepdims=True)
        bn_ref[pl.ds(j * FT, FT), :] = rows / jnp.sqrt(
            jnp.maximum(n2, 1e-24))
        return 0
    lax.fori_loop(0, nbt, nrm, 0)

    s1 = jnp.float32(0.0)
    s3 = jnp.float32(0.0)
    for r in range(nft):
        fr = fn_ref[r * FT:(r + 1) * FT, :]
        w_r = wgt[:, r * FT:(r + 1) * FT]
        amin = jnp.full((FT, 1), jnp.inf, jnp.float32)
        for q in range(nft):
            g = _dotT(fr, fn_ref[q * FT:(q + 1) * FT, :])
            wm = wgt[:, q * FT:(q + 1) * FT] > 0
            amin = jnp.minimum(amin, jnp.min(
                jnp.where(wm, g, jnp.inf), axis=1, keepdims=True))
        s1 = s1 + _rowsum(w_r, jnp.maximum(TH_SIMILAR_MIN - amin, 0.0))

        def fb(j, cmax):
            g = _dotT(fr, bn_ref[pl.ds(j * FT, FT), :])
            lanes = lax.broadcasted_iota(jnp.int32, (1, FT), 1) + j * FT
            bm = lanes < cnt
            return jnp.maximum(cmax, jnp.max(
                jnp.where(bm, g, -jnp.inf), axis=1, keepdims=True))
        cmax = lax.fori_loop(0, nbt, fb,
                             jnp.full((FT, 1), -jnp.inf, jnp.float32))
        s3 = s3 + _rowsum(w_r, jnp.maximum(cmax - TH_DIFFERENT_MAX, 0.0))

    def brow(rj, s24):
        s2, s4 = s24
        br = bn_ref[pl.ds(rj * FT, FT), :]
        rmask = (lax.broadcasted_iota(jnp.int32, (FT, 1), 0)
                 + rj * FT) < cnt
        dmax = jnp.full((FT, 1), -jnp.inf, jnp.float32)
        for q in range(nft):
            g = _dotT(br, fn_ref[q * FT:(q + 1) * FT, :])
            wm = wgt[:, q * FT:(q + 1) * FT] > 0
            dmax = jnp.maximum(dmax, jnp.max(
                jnp.where(wm, g, -jnp.inf), axis=1, keepdims=True))
        rel4 = jnp.where(rmask,
                         jnp.maximum(dmax - TH_DIFFERENT_MAX, 0.0), 0.0)
        s4 = s4 + jnp.sum(rel4)

        def bb(cj, bmin):
            g = _dotT(br, bn_ref[pl.ds(cj * FT, FT), :])
            lanes = lax.broadcasted_iota(jnp.int32, (1, FT), 1) + cj * FT
            bm = lanes < cnt
            return jnp.minimum(bmin, jnp.min(
                jnp.where(bm, g, jnp.inf), axis=1, keepdims=True))
        bmin = lax.fori_loop(0, nbt, bb,
                             jnp.full((FT, 1), jnp.inf, jnp.float32))
        rel2 = jnp.where(rmask,
                         jnp.maximum(TH_SIMILAR_MIN - bmin, 0.0), 0.0)
        s2 = s2 + jnp.sum(rel2)
        return s2, s4

    s2, s4 = lax.fori_loop(0, nbt, brow,
                           (jnp.float32(0.0), jnp.float32(0.0)))

    loss_b = (s1 + s3) / c_fg + (s2 + s4) / c_bg
    valid = cnt > 0
    loss_ref[...] = jnp.zeros((1, 8, 128), jnp.float32) + jnp.where(
        valid, loss_b, 0.0)
    val_ref[...] = jnp.zeros((1, 8, 128), jnp.float32) + jnp.where(
        valid, 1.0, 0.0)


def _main(fgbuf, label, bgbuf, counts):
    b, nfg, d = fgbuf.shape
    t = bgbuf.shape[1]
    c = label.shape[1]
    lab3 = label.reshape(b, 1, c)
    cnt3 = counts.reshape(b, 1, 16)
    return pl.pallas_call(
        _main_body,
        grid=(b,),
        in_specs=[
            pl.BlockSpec((1, 1, 16), lambda i: (i, 0, 0),
                         memory_space=pltpu.SMEM),
            pl.BlockSpec((1, nfg, d), lambda i: (i, 0, 0)),
            pl.BlockSpec((1, 1, c), lambda i: (i, 0, 0)),
            pl.BlockSpec((1, t, d), lambda i: (i, 0, 0)),
        ],
        out_specs=[
            pl.BlockSpec((1, 8, 128), lambda i: (i, 0, 0)),
            pl.BlockSpec((1, 8, 128), lambda i: (i, 0, 0)),
        ],
        out_shape=[
            jax.ShapeDtypeStruct((b, 8, 128), jnp.float32),
            jax.ShapeDtypeStruct((b, 8, 128), jnp.float32),
        ],
        scratch_shapes=[
            pltpu.VMEM((nfg, d), jnp.float32),
            pltpu.VMEM((t, d), jnp.float32),
        ],
    )(cnt3, fgbuf, lab3, bgbuf)


def kernel(embeddings, click_labels, cas_supp, label, cls_num):
    b, d, t = embeddings.shape
    c = cas_supp.shape[1]

    sel = _topk_sel(cas_supp)
    etr = jnp.swapaxes(embeddings, 1, 2)
    cls_arr = jnp.zeros((16,), jnp.int32) + jnp.asarray(cls_num, jnp.int32)

    fgbuf, bgbuf, counts = _sc_gather(sel, click_labels, etr, cls_arr)
    lossm, validf = _main(fgbuf, label, bgbuf, counts)
    return jnp.sum(lossm[:, 0, 0]) / jnp.sum(validf[:, 0, 0])


# speedup vs baseline: 5.8872x; 2.2911x over previous
import functools

import jax
import jax.numpy as jnp
from jax import lax
from jax.experimental import pallas as pl
from jax.experimental.pallas import tpu as pltpu
from jax.experimental.pallas import tpu_sc as plsc

K = 64
TH_SIMILAR_MIN = 0.5
TH_DIFFERENT_MAX = 0.1
_PREC = lax.Precision.DEFAULT
FT = 256


def _topk_body(cas_ref, sel_ref):
    c, t = cas_ref.shape[1], cas_ref.shape[2]
    cas = cas_ref[0]
    col = lax.broadcasted_iota(jnp.int32, (c, t), 1)
    one = jnp.ones((c, t), jnp.int32)
    zero = jnp.zeros((c, t), jnp.int32)

    bits = lax.bitcast_convert_type(cas, jnp.int32)
    key = bits ^ (lax.shift_right_arithmetic(bits, 31) & 0x7FFFFFFF)

    def cnt_ge(cand):
        return jnp.sum(jnp.where(key >= cand, one, zero),
                       axis=1, keepdims=True)

    pos = cnt_ge(jnp.zeros((c, 1), jnp.int32)) >= K
    tt = jnp.where(pos, jnp.int32(0), jnp.int32(-2147483648))
    for bit in range(30, -1, -1):
        cand = tt | (1 << bit)
        tt = jnp.where(cnt_ge(cand) >= K, cand, tt)

    gt = key > tt
    eq = key == tt
    n_gt = jnp.sum(jnp.where(gt, one, zero), axis=1, keepdims=True)
    need = K - n_gt

    nbits = max(1, (t - 1).bit_length() - 1)
    bb = jnp.zeros((c, 1), jnp.int32)
    for bit in range(nbits, -1, -1):
        cand = bb | (1 << bit)
        cntb = jnp.sum(jnp.where(eq & (col < cand), one, zero),
                       axis=1, keepdims=True)
        bb = jnp.where(cntb < need, cand, bb)

    sel = gt | (eq & (col < (bb + 1)))
    sel_ref[0] = jnp.where(sel, 1.0, 0.0)


def _topk_sel(cas_supp):
    b, c, t = cas_supp.shape
    return pl.pallas_call(
        _topk_body,
        grid=(b,),
        in_specs=[pl.BlockSpec((1, c, t), lambda i: (i, 0, 0))],
        out_specs=pl.BlockSpec((1, c, t), lambda i: (i, 0, 0)),
        out_shape=jax.ShapeDtypeStruct((b, c, t), jnp.float32),
    )(cas_supp)


def _compact(src_ref, idx_ref, match_fn, t):
    def body(i, cursor):
        v = src_ref[pl.ds(i * 16, 16)]
        m = match_fn(v)
        mi = jnp.where(m, jnp.full((16,), 1, jnp.int32),
                       jnp.zeros((16,), jnp.int32))
        excl = plsc.cumsum(mi) - mi
        idx = lax.iota(jnp.int32, 16) + i * 16
        plsc.store_scatter(idx_ref, [excl + cursor], idx, mask=m)
        return cursor + jnp.sum(mi)
    return lax.fori_loop(0, t // 16, body, jnp.int32(0))


def _sc_gather(sel, click, etr, cls_arr):
    b, c, t = sel.shape
    d = etr.shape[2]
    nfg = c * K

    mesh = plsc.VectorSubcoreMesh(core_axis_name="c", subcore_axis_name="s")

    @functools.partial(
        pl.kernel, mesh=mesh,
        compiler_params=pltpu.CompilerParams(needs_layout_passes=False),
        out_type=[
            jax.ShapeDtypeStruct((b, nfg, d), jnp.float32),
            jax.ShapeDtypeStruct((b, t, d), jnp.float32),
            jax.ShapeDtypeStruct((b, 16), jnp.int32),
        ],
        scratch_types=[
            pltpu.VMEM((t,), jnp.float32),
            pltpu.VMEM((t,), jnp.int32),
            pltpu.VMEM((t + 32,), jnp.int32),
            pltpu.VMEM((K,), jnp.int32),
            pltpu.VMEM((128,), jnp.int32),
            pltpu.VMEM((K, 128), jnp.float32),
            pltpu.VMEM((128, 128), jnp.float32),
            pltpu.VMEM((16,), jnp.int32),
            pltpu.SemaphoreType.DMA,
        ],
    )
    def sck(sel_h, click_h, etr_h, cls_h,
            fg_h, bg_h, cnt_h,
            selv, clickv, idxv, idxfg, idxch, rowsf, rows, sv, sem):
        wid = lax.axis_index("s") * 2 + lax.axis_index("c")

        for i in range(10):
            task = wid * 10 + i
            bb = task // c
            cc = task - bb * c
            pltpu.sync_copy(sel_h.at[bb].at[cc], selv)
            n = _compact(selv, idxfg, lambda v: v > 0.5, t)
            del n
            pltpu.async_copy(etr_h.at[bb].at[idxfg], rowsf, sem).wait()
            pltpu.sync_copy(rowsf, fg_h.at[bb].at[pl.ds(cc * K, K)])

        @pl.when(wid < b)
        def _():
            bb = wid
            pltpu.sync_copy(cls_h, sv)
            clsv = sv[...]
            pltpu.sync_copy(click_h.at[bb], clickv)

            def zero(i, _):
                idxv[pl.ds(i * 16, 16)] = jnp.zeros((16,), jnp.int32)
                return 0
            lax.fori_loop(0, (t + 32) // 16, zero, 0)

            def body(i, cursor):
                v = clickv[pl.ds(i * 16, 16)]
                m = v == clsv
                mi = jnp.where(m, jnp.full((16,), 1, jnp.int32),
                               jnp.zeros((16,), jnp.int32))
                excl = plsc.cumsum(mi) - mi
                idx = lax.iota(jnp.int32, 16) + i * 16
                plsc.store_scatter(idxv, [excl + cursor], idx, mask=m)
                return cursor + jnp.sum(mi)
            cnt = lax.fori_loop(0, t // 16, body, jnp.int32(0))

            sv[...] = jnp.zeros((16,), jnp.int32) + cnt
            pltpu.sync_copy(sv, cnt_h.at[bb])

            nch = (cnt + 127) // 128

            def gat(j, _):
                for k in range(8):
                    idxch[pl.ds(k * 16, 16)] = idxv[
                        pl.ds(j * 128 + k * 16, 16)]
                pltpu.async_copy(etr_h.at[bb].at[idxch], rows, sem).wait()
                pltpu.sync_copy(rows, bg_h.at[bb].at[pl.ds(j * 128, 128)])
                return 0
            lax.fori_loop(0, nch, gat, 0)

    return sck(sel, click, etr, cls_arr)


def _dotT(a, bmat):
    return lax.dot_general(a, bmat, (((1,), (1,)), ((), ())),
                           preferred_element_type=jnp.float32,
                           precision=_PREC)


def _rowsum(rowvec, colvec):
    return lax.dot_general(rowvec, colvec, (((1,), (0,)), ((), ())),
                           preferred_element_type=jnp.float32,
                           precision=_PREC)[0, 0]


def _main_body(cnt_ref, fg_ref, lab_ref, bg_ref, loss_ref, val_ref,
               fn_ref, bn_ref):
    nfg, d = fn_ref.shape
    t = bn_ref.shape[0]
    c = lab_ref.shape[2]
    nft = nfg // FT

    cnt = cnt_ref[0, 0, 0]
    c_bg = cnt.astype(jnp.float32)
    labf = (lab_ref[0] == 1).astype(jnp.float32)
    rsel = (lax.broadcasted_iota(jnp.int32, (c, nfg), 1) // K
            == lax.broadcasted_iota(jnp.int32, (c, nfg), 0)
            ).astype(jnp.float32)
    wgt = lax.dot_general(labf, rsel, (((1,), (0,)), ((), ())),
                          preferred_element_type=jnp.float32,
                          precision=_PREC)
    c_fg = jnp.sum(wgt)
    nbt = lax.div(cnt + (FT - 1), FT)

    for i in range(nft):
        rows = fg_ref[0, i * FT:(i + 1) * FT, :]
        n2 = jnp.sum(rows * rows, axis=1, keepdims=True)
        fn_ref[i * FT:(i + 1) * FT, :] = rows / jnp.sqrt(
            jnp.maximum(n2, 1e-24))

    def nrm(j, _):
        rows = bg_ref[0, pl.ds(j * FT, FT), :]
        n2 = jnp.sum(rows * rows, axis=1, keepdims=True)
        bn_ref[pl.ds(j * FT, FT), :] = rows / jnp.sqrt(
            jnp.maximum(n2, 1e-24))
        return 0
    lax.fori_loop(0, nbt, nrm, 0)

    s1 = jnp.float32(0.0)
    s3 = jnp.float32(0.0)
    for r in range(nft):
        fr = fn_ref[r * FT:(r + 1) * FT, :]
        w_r = wgt[:, r * FT:(r + 1) * FT]
        amin = jnp.full((FT, 1), jnp.inf, jnp.float32)
        for q in range(nft):
            g = _dotT(fr, fn_ref[q * FT:(q + 1) * FT, :])
            wm = wgt[:, q * FT:(q + 1) * FT] > 0
            amin = jnp.minimum(amin, jnp.min(
                jnp.where(wm, g, jnp.inf), axis=1, keepdims=True))
        s1 = s1 + _rowsum(w_r, jnp.maximum(TH_SIMILAR_MIN - amin, 0.0))

        def fb(j, cmax):
            g = _dotT(fr, bn_ref[pl.ds(j * FT, FT), :])
            lanes = lax.broadcasted_iota(jnp.int32, (1, FT), 1) + j * FT
            bm = lanes < cnt
            return jnp.maximum(cmax, jnp.max(
                jnp.where(bm, g, -jnp.inf), axis=1, keepdims=True))
        cmax = lax.fori_loop(0, nbt, fb,
                             jnp.full((FT, 1), -jnp.inf, jnp.float32))
        s3 = s3 + _rowsum(w_r, jnp.maximum(cmax - TH_DIFFERENT_MAX, 0.0))

    def brow(rj, s24):
        s2, s4 = s24
        br = bn_ref[pl.ds(rj * FT, FT), :]
        rmask = (lax.broadcasted_iota(jnp.int32, (FT, 1), 0)
                 + rj * FT) < cnt
        dmax = jnp.full((FT, 1), -jnp.inf, jnp.float32)
        for q in range(nft):
            g = _dotT(br, fn_ref[q * FT:(q + 1) * FT, :])
            wm = wgt[:, q * FT:(q + 1) * FT] > 0
            dmax = jnp.maximum(dmax, jnp.max(
                jnp.where(wm, g, -jnp.inf), axis=1, keepdims=True))
        rel4 = jnp.where(rmask,
                         jnp.maximum(dmax - TH_DIFFERENT_MAX, 0.0), 0.0)
        s4 = s4 + jnp.sum(rel4)

        def bb(cj, bmin):
            g = _dotT(br, bn_ref[pl.ds(cj * FT, FT), :])
            lanes = lax.broadcasted_iota(jnp.int32, (1, FT), 1) + cj * FT
            bm = lanes < cnt
            return jnp.minimum(bmin, jnp.min(
                jnp.where(bm, g, jnp.inf), axis=1, keepdims=True))
        bmin = lax.fori_loop(0, nbt, bb,
                             jnp.full((FT, 1), jnp.inf, jnp.float32))
        rel2 = jnp.where(rmask,
                         jnp.maximum(TH_SIMILAR_MIN - bmin, 0.0), 0.0)
        s2 = s2 + jnp.sum(rel2)
        return s2, s4

    s2, s4 = lax.fori_loop(0, nbt, brow,
                           (jnp.float32(0.0), jnp.float32(0.0)))

    loss_b = (s1 + s3) / c_fg + (s2 + s4) / c_bg
    valid = cnt > 0
    loss_ref[...] = jnp.zeros((1, 8, 128), jnp.float32) + jnp.where(
        valid, loss_b, 0.0)
    val_ref[...] = jnp.zeros((1, 8, 128), jnp.float32) + jnp.where(
        valid, 1.0, 0.0)


def _main(fgbuf, label, bgbuf, counts):
    b, nfg, d = fgbuf.shape
    t = bgbuf.shape[1]
    c = label.shape[1]
    lab3 = label.reshape(b, 1, c)
    cnt3 = counts.reshape(b, 1, 16)
    return pl.pallas_call(
        _main_body,
        grid=(b,),
        in_specs=[
            pl.BlockSpec((1, 1, 16), lambda i: (i, 0, 0),
                         memory_space=pltpu.SMEM),
            pl.BlockSpec((1, nfg, d), lambda i: (i, 0, 0)),
            pl.BlockSpec((1, 1, c), lambda i: (i, 0, 0)),
            pl.BlockSpec((1, t, d), lambda i: (i, 0, 0)),
        ],
        out_specs=[
            pl.BlockSpec((1, 8, 128), lambda i: (i, 0, 0)),
            pl.BlockSpec((1, 8, 128), lambda i: (i, 0, 0)),
        ],
        out_shape=[
            jax.ShapeDtypeStruct((b, 8, 128), jnp.float32),
            jax.ShapeDtypeStruct((b, 8, 128), jnp.float32),
        ],
        scratch_shapes=[
            pltpu.VMEM((nfg, d), jnp.float32),
            pltpu.VMEM((t, d), jnp.float32),
        ],
    )(cnt3, fgbuf, lab3, bgbuf)


def kernel(embeddings, click_labels, cas_supp, label, cls_num):
    b, d, t = embeddings.shape
    c = cas_supp.shape[1]

    sel = _topk_sel(cas_supp)
    etr = jnp.swapaxes(embeddings, 1, 2)
    cls_arr = jnp.zeros((16,), jnp.int32) + jnp.asarray(cls_num, jnp.int32)

    fgbuf, bgbuf, counts = _sc_gather(sel, click_labels, etr, cls_arr)
    lossm, validf = _main(fgbuf, label, bgbuf, counts)
    return jnp.sum(lossm[:, 0, 0]) / jnp.sum(validf[:, 0, 0])


# speedup vs baseline: 5.8926x; 1.0009x over previous
import functools

import jax
import jax.numpy as jnp
from jax import lax
from jax.experimental import pallas as pl
from jax.experimental.pallas import tpu as pltpu
from jax.experimental.pallas import tpu_sc as plsc

K = 64
TH_SIMILAR_MIN = 0.5
TH_DIFFERENT_MAX = 0.1
_PREC = lax.Precision.DEFAULT
FT = 256


def _topk_body(cas_ref, sel_ref):
    c, t = cas_ref.shape[1], cas_ref.shape[2]
    cas = cas_ref[0]
    col = lax.broadcasted_iota(jnp.int32, (c, t), 1)
    one = jnp.ones((c, t), jnp.int32)
    zero = jnp.zeros((c, t), jnp.int32)

    bits = lax.bitcast_convert_type(cas, jnp.int32)
    key = bits ^ (lax.shift_right_arithmetic(bits, 31) & 0x7FFFFFFF)

    def cnt_ge(cand):
        return jnp.sum(jnp.where(key >= cand, one, zero),
                       axis=1, keepdims=True)

    pos = cnt_ge(jnp.zeros((c, 1), jnp.int32)) >= K
    tt = jnp.where(pos, jnp.int32(0), jnp.int32(-2147483648))
    for bit in range(30, -1, -1):
        cand = tt | (1 << bit)
        tt = jnp.where(cnt_ge(cand) >= K, cand, tt)

    gt = key > tt
    eq = key == tt
    n_gt = jnp.sum(jnp.where(gt, one, zero), axis=1, keepdims=True)
    need = K - n_gt

    nbits = max(1, (t - 1).bit_length() - 1)
    bb = jnp.zeros((c, 1), jnp.int32)
    for bit in range(nbits, -1, -1):
        cand = bb | (1 << bit)
        cntb = jnp.sum(jnp.where(eq & (col < cand), one, zero),
                       axis=1, keepdims=True)
        bb = jnp.where(cntb < need, cand, bb)

    sel = gt | (eq & (col < (bb + 1)))
    sel_ref[0] = jnp.where(sel, 1.0, 0.0)


def _topk_sel(cas_supp):
    b, c, t = cas_supp.shape
    return pl.pallas_call(
        _topk_body,
        grid=(b,),
        in_specs=[pl.BlockSpec((1, c, t), lambda i: (i, 0, 0))],
        out_specs=pl.BlockSpec((1, c, t), lambda i: (i, 0, 0)),
        out_shape=jax.ShapeDtypeStruct((b, c, t), jnp.float32),
    )(cas_supp)


def _compact(src_ref, idx_ref, match_fn, t):
    one16 = jnp.full((16,), 1, jnp.int32)
    zero16 = jnp.zeros((16,), jnp.int32)

    def body(i, cursor_v):
        for u in range(2):
            j = i * 2 + u
            v = src_ref[pl.ds(j * 16, 16)]
            m = match_fn(v)
            mi = jnp.where(m, one16, zero16)
            excl = plsc.cumsum(mi) - mi
            idx = lax.iota(jnp.int32, 16) + j * 16
            plsc.store_scatter(idx_ref, [excl + cursor_v], idx, mask=m)
            cursor_v = cursor_v + plsc.all_reduce_population_count(m)
        return cursor_v
    cv = lax.fori_loop(0, t // 32, body, jnp.zeros((16,), jnp.int32))
    return cv[0]


def _sc_gather(sel, click, etr, cls_arr):
    b, c, t = sel.shape
    d = etr.shape[2]
    nfg = c * K

    mesh = plsc.VectorSubcoreMesh(core_axis_name="c", subcore_axis_name="s")

    @functools.partial(
        pl.kernel, mesh=mesh,
        compiler_params=pltpu.CompilerParams(needs_layout_passes=False),
        out_type=[
            jax.ShapeDtypeStruct((b, nfg, d), jnp.float32),
            jax.ShapeDtypeStruct((b, t, d), jnp.float32),
            jax.ShapeDtypeStruct((b, 16), jnp.int32),
        ],
        scratch_types=[
            pltpu.VMEM((t,), jnp.float32),
            pltpu.VMEM((t,), jnp.int32),
            pltpu.VMEM((t + 32,), jnp.int32),
            pltpu.VMEM((K,), jnp.int32),
            pltpu.VMEM((128,), jnp.int32),
            pltpu.VMEM((K, 128), jnp.float32),
            pltpu.VMEM((128, 128), jnp.float32),
            pltpu.VMEM((16,), jnp.int32),
            pltpu.SemaphoreType.DMA,
        ],
    )
    def sck(sel_h, click_h, etr_h, cls_h,
            fg_h, bg_h, cnt_h,
            selv, clickv, idxv, idxfg, idxch, rowsf, rows, sv, sem):
        wid = lax.axis_index("s") * 2 + lax.axis_index("c")

        for i in range(10):
            task = wid * 10 + i
            bb = task // c
            cc = task - bb * c
            pltpu.sync_copy(sel_h.at[bb].at[cc], selv)
            n = _compact(selv, idxfg, lambda v: v > 0.5, t)
            del n
            pltpu.async_copy(etr_h.at[bb].at[idxfg], rowsf, sem).wait()
            pltpu.sync_copy(rowsf, fg_h.at[bb].at[pl.ds(cc * K, K)])

        @pl.when(wid < b)
        def _():
            bb = wid
            pltpu.sync_copy(cls_h, sv)
            clsv = sv[...]
            pltpu.sync_copy(click_h.at[bb], clickv)

            def zero(i, _):
                idxv[pl.ds(i * 16, 16)] = jnp.zeros((16,), jnp.int32)
                return 0
            lax.fori_loop(0, (t + 32) // 16, zero, 0)

            cnt = _compact(clickv, idxv, lambda v: v == clsv, t)

            sv[...] = jnp.zeros((16,), jnp.int32) + cnt
            pltpu.sync_copy(sv, cnt_h.at[bb])

            nch = (cnt + 127) // 128

            def gat(j, _):
                for k in range(8):
                    idxch[pl.ds(k * 16, 16)] = idxv[
                        pl.ds(j * 128 + k * 16, 16)]
                pltpu.async_copy(etr_h.at[bb].at[idxch], rows, sem).wait()
                pltpu.sync_copy(rows, bg_h.at[bb].at[pl.ds(j * 128, 128)])
                return 0
            lax.fori_loop(0, nch, gat, 0)

    return sck(sel, click, etr, cls_arr)


def _dotT(a, bmat):
    return lax.dot_general(a, bmat, (((1,), (1,)), ((), ())),
                           preferred_element_type=jnp.float32,
                           precision=_PREC)


def _rowsum(rowvec, colvec):
    return lax.dot_general(rowvec, colvec, (((1,), (0,)), ((), ())),
                           preferred_element_type=jnp.float32,
                           precision=_PREC)[0, 0]


def _main_body(cnt_ref, fg_ref, lab_ref, bg_ref, loss_ref, val_ref,
               fn_ref, bn_ref):
    nfg, d = fn_ref.shape
    t = bn_ref.shape[0]
    c = lab_ref.shape[2]
    nft = nfg // FT

    cnt = cnt_ref[0, 0, 0]
    c_bg = cnt.astype(jnp.float32)
    labf = (lab_ref[0] == 1).astype(jnp.float32)
    rsel = (lax.broadcasted_iota(jnp.int32, (c, nfg), 1) // K
            == lax.broadcasted_iota(jnp.int32, (c, nfg), 0)
            ).astype(jnp.float32)
    wgt = lax.dot_general(labf, rsel, (((1,), (0,)), ((), ())),
                          preferred_element_type=jnp.float32,
                          precision=_PREC)
    c_fg = jnp.sum(wgt)
    nbt = lax.div(cnt + (FT - 1), FT)

    for i in range(nft):
        rows = fg_ref[0, i * FT:(i + 1) * FT, :]
        n2 = jnp.sum(rows * rows, axis=1, keepdims=True)
        fn_ref[i * FT:(i + 1) * FT, :] = rows / jnp.sqrt(
            jnp.maximum(n2, 1e-24))

    def nrm(j, _):
        rows = bg_ref[0, pl.ds(j * FT, FT), :]
        n2 = jnp.sum(rows * rows, axis=1, keepdims=True)
        bn_ref[pl.ds(j * FT, FT), :] = rows / jnp.sqrt(
            jnp.maximum(n2, 1e-24))
        return 0
    lax.fori_loop(0, nbt, nrm, 0)

    s1 = jnp.float32(0.0)
    s3 = jnp.float32(0.0)
    for r in range(nft):
        fr = fn_ref[r * FT:(r + 1) * FT, :]
        w_r = wgt[:, r * FT:(r + 1) * FT]
        amin = jnp.full((FT, 1), jnp.inf, jnp.float32)
        for q in range(nft):
            g = _dotT(fr, fn_ref[q * FT:(q + 1) * FT, :])
            wm = wgt[:, q * FT:(q + 1) * FT] > 0
            amin = jnp.minimum(amin, jnp.min(
                jnp.where(wm, g, jnp.inf), axis=1, keepdims=True))
        s1 = s1 + _rowsum(w_r, jnp.maximum(TH_SIMILAR_MIN - amin, 0.0))

        def fb(j, cmax):
            g = _dotT(fr, bn_ref[pl.ds(j * FT, FT), :])
            lanes = lax.broadcasted_iota(jnp.int32, (1, FT), 1) + j * FT
            bm = lanes < cnt
            return jnp.maximum(cmax, jnp.max(
                jnp.where(bm, g, -jnp.inf), axis=1, keepdims=True))
        cmax = lax.fori_loop(0, nbt, fb,
                             jnp.full((FT, 1), -jnp.inf, jnp.float32))
        s3 = s3 + _rowsum(w_r, jnp.maximum(cmax - TH_DIFFERENT_MAX, 0.0))

    def brow(rj, s24):
        s2, s4 = s24
        br = bn_ref[pl.ds(rj * FT, FT), :]
        rmask = (lax.broadcasted_iota(jnp.int32, (FT, 1), 0)
                 + rj * FT) < cnt
        dmax = jnp.full((FT, 1), -jnp.inf, jnp.float32)
        for q in range(nft):
            g = _dotT(br, fn_ref[q * FT:(q + 1) * FT, :])
            wm = wgt[:, q * FT:(q + 1) * FT] > 0
            dmax = jnp.maximum(dmax, jnp.max(
                jnp.where(wm, g, -jnp.inf), axis=1, keepdims=True))
        rel4 = jnp.where(rmask,
                         jnp.maximum(dmax - TH_DIFFERENT_MAX, 0.0), 0.0)
        s4 = s4 + jnp.sum(rel4)

        def bb(cj, bmin):
            g = _dotT(br, bn_ref[pl.ds(cj * FT, FT), :])
            lanes = lax.broadcasted_iota(jnp.int32, (1, FT), 1) + cj * FT
            bm = lanes < cnt
            return jnp.minimum(bmin, jnp.min(
                jnp.where(bm, g, jnp.inf), axis=1, keepdims=True))
        bmin = lax.fori_loop(0, nbt, bb,
                             jnp.full((FT, 1), jnp.inf, jnp.float32))
        rel2 = jnp.where(rmask,
                         jnp.maximum(TH_SIMILAR_MIN - bmin, 0.0), 0.0)
        s2 = s2 + jnp.sum(rel2)
        return s2, s4

    s2, s4 = lax.fori_loop(0, nbt, brow,
                           (jnp.float32(0.0), jnp.float32(0.0)))

    loss_b = (s1 + s3) / c_fg + (s2 + s4) / c_bg
    valid = cnt > 0
    loss_ref[...] = jnp.zeros((1, 8, 128), jnp.float32) + jnp.where(
        valid, loss_b, 0.0)
    val_ref[...] = jnp.zeros((1, 8, 128), jnp.float32) + jnp.where(
        valid, 1.0, 0.0)


def _main(fgbuf, label, bgbuf, counts):
    b, nfg, d = fgbuf.shape
    t = bgbuf.shape[1]
    c = label.shape[1]
    lab3 = label.reshape(b, 1, c)
    cnt3 = counts.reshape(b, 1, 16)
    return pl.pallas_call(
        _main_body,
        grid=(b,),
        in_specs=[
            pl.BlockSpec((1, 1, 16), lambda i: (i, 0, 0),
                         memory_space=pltpu.SMEM),
            pl.BlockSpec((1, nfg, d), lambda i: (i, 0, 0)),
            pl.BlockSpec((1, 1, c), lambda i: (i, 0, 0)),
            pl.BlockSpec((1, t, d), lambda i: (i, 0, 0)),
        ],
        out_specs=[
            pl.BlockSpec((1, 8, 128), lambda i: (i, 0, 0)),
            pl.BlockSpec((1, 8, 128), lambda i: (i, 0, 0)),
        ],
        out_shape=[
            jax.ShapeDtypeStruct((b, 8, 128), jnp.float32),
            jax.ShapeDtypeStruct((b, 8, 128), jnp.float32),
        ],
        scratch_shapes=[
            pltpu.VMEM((nfg, d), jnp.float32),
            pltpu.VMEM((t, d), jnp.float32),
        ],
    )(cnt3, fgbuf, lab3, bgbuf)


def kernel(embeddings, click_labels, cas_supp, label, cls_num):
    b, d, t = embeddings.shape
    c = cas_supp.shape[1]

    sel = _topk_sel(cas_supp)
    etr = jnp.swapaxes(embeddings, 1, 2)
    cls_arr = jnp.zeros((16,), jnp.int32) + jnp.asarray(cls_num, jnp.int32)

    fgbuf, bgbuf, counts = _sc_gather(sel, click_labels, etr, cls_arr)
    lossm, validf = _main(fgbuf, label, bgbuf, counts)
    return jnp.sum(lossm[:, 0, 0]) / jnp.sum(validf[:, 0, 0])


# speedup vs baseline: 5.9027x; 1.0017x over previous
import functools

import jax
import jax.numpy as jnp
from jax import lax
from jax.experimental import pallas as pl
from jax.experimental.pallas import tpu as pltpu
from jax.experimental.pallas import tpu_sc as plsc

K = 64
TH_SIMILAR_MIN = 0.5
TH_DIFFERENT_MAX = 0.1
_PREC = lax.Precision.DEFAULT
FT = 256


def _topk_body(cas_ref, sel_ref):
    c, t = cas_ref.shape[1], cas_ref.shape[2]
    cas = cas_ref[0]
    col = lax.broadcasted_iota(jnp.int32, (c, t), 1)
    one = jnp.ones((c, t), jnp.int32)
    zero = jnp.zeros((c, t), jnp.int32)

    bits = lax.bitcast_convert_type(cas, jnp.int32)
    key = bits ^ (lax.shift_right_arithmetic(bits, 31) & 0x7FFFFFFF)

    def cnt_ge(cand):
        return jnp.sum(jnp.where(key >= cand, one, zero),
                       axis=1, keepdims=True)

    pos = cnt_ge(jnp.zeros((c, 1), jnp.int32)) >= K
    tt = jnp.where(pos, jnp.int32(0), jnp.int32(-2147483648))
    cand = tt | (1 << 30)
    tt = jnp.where(cnt_ge(cand) >= K, cand, tt)
    for hb in range(29, 0, -2):
        b3 = tt | (3 << (hb - 1))
        b2 = tt | (1 << hb)
        b1 = tt | (1 << (hb - 1))
        c3 = cnt_ge(b3) >= K
        c2 = cnt_ge(b2) >= K
        c1 = cnt_ge(b1) >= K
        tt = jnp.where(c3, b3, jnp.where(c2, b2, jnp.where(c1, b1, tt)))

    gt = key > tt
    eq = key == tt
    n_gt = jnp.sum(jnp.where(gt, one, zero), axis=1, keepdims=True)
    need = K - n_gt

    def cnt_lt(cand):
        return jnp.sum(jnp.where(eq & (col < cand), one, zero),
                       axis=1, keepdims=True)

    nb = (t - 1).bit_length()
    bb = jnp.zeros((c, 1), jnp.int32)
    if nb % 2 == 1:
        cand = bb | (1 << (nb - 1))
        bb = jnp.where(cnt_lt(cand) < need, cand, bb)
        nb -= 1
    for hb in range(nb - 1, 0, -2):
        b3 = bb | (3 << (hb - 1))
        b2 = bb | (1 << hb)
        b1 = bb | (1 << (hb - 1))
        p3 = cnt_lt(b3) < need
        p2 = cnt_lt(b2) < need
        p1 = cnt_lt(b1) < need
        bb = jnp.where(p3, b3, jnp.where(p2, b2, jnp.where(p1, b1, bb)))

    sel = gt | (eq & (col < (bb + 1)))
    sel_ref[0] = jnp.where(sel, 1.0, 0.0)


def _topk_sel(cas_supp):
    b, c, t = cas_supp.shape
    return pl.pallas_call(
        _topk_body,
        grid=(b,),
        in_specs=[pl.BlockSpec((1, c, t), lambda i: (i, 0, 0))],
        out_specs=pl.BlockSpec((1, c, t), lambda i: (i, 0, 0)),
        out_shape=jax.ShapeDtypeStruct((b, c, t), jnp.float32),
    )(cas_supp)


def _compact(src_ref, idx_ref, match_fn, t):
    one16 = jnp.full((16,), 1, jnp.int32)
    zero16 = jnp.zeros((16,), jnp.int32)

    def body(i, cursor_v):
        for u in range(2):
            j = i * 2 + u
            v = src_ref[pl.ds(j * 16, 16)]
            m = match_fn(v)
            mi = jnp.where(m, one16, zero16)
            excl = plsc.cumsum(mi) - mi
            idx = lax.iota(jnp.int32, 16) + j * 16
            plsc.store_scatter(idx_ref, [excl + cursor_v], idx, mask=m)
            cursor_v = cursor_v + plsc.all_reduce_population_count(m)
        return cursor_v
    cv = lax.fori_loop(0, t // 32, body, jnp.zeros((16,), jnp.int32))
    return cv[0]


def _sc_gather(sel, click, etr, cls_arr):
    b, c, t = sel.shape
    d = etr.shape[2]
    nfg = c * K

    mesh = plsc.VectorSubcoreMesh(core_axis_name="c", subcore_axis_name="s")

    @functools.partial(
        pl.kernel, mesh=mesh,
        compiler_params=pltpu.CompilerParams(needs_layout_passes=False),
        out_type=[
            jax.ShapeDtypeStruct((b, nfg, d), jnp.float32),
            jax.ShapeDtypeStruct((b, t, d), jnp.float32),
            jax.ShapeDtypeStruct((b, 16), jnp.int32),
        ],
        scratch_types=[
            pltpu.VMEM((t,), jnp.float32),
            pltpu.VMEM((t,), jnp.int32),
            pltpu.VMEM((t + 32,), jnp.int32),
            pltpu.VMEM((K,), jnp.int32),
            pltpu.VMEM((128,), jnp.int32),
            pltpu.VMEM((K, 128), jnp.float32),
            pltpu.VMEM((128, 128), jnp.float32),
            pltpu.VMEM((16,), jnp.int32),
            pltpu.SemaphoreType.DMA,
        ],
    )
    def sck(sel_h, click_h, etr_h, cls_h,
            fg_h, bg_h, cnt_h,
            selv, clickv, idxv, idxfg, idxch, rowsf, rows, sv, sem):
        wid = lax.axis_index("s") * 2 + lax.axis_index("c")

        for i in range(10):
            task = wid * 10 + i
            bb = task // c
            cc = task - bb * c
            pltpu.sync_copy(sel_h.at[bb].at[cc], selv)
            n = _compact(selv, idxfg, lambda v: v > 0.5, t)
            del n
            pltpu.async_copy(etr_h.at[bb].at[idxfg], rowsf, sem).wait()
            pltpu.sync_copy(rowsf, fg_h.at[bb].at[pl.ds(cc * K, K)])

        @pl.when(wid < b)
        def _():
            bb = wid
            pltpu.sync_copy(cls_h, sv)
            clsv = sv[...]
            pltpu.sync_copy(click_h.at[bb], clickv)

            def zero(i, _):
                idxv[pl.ds(i * 16, 16)] = jnp.zeros((16,), jnp.int32)
                return 0
            lax.fori_loop(0, (t + 32) // 16, zero, 0)

            cnt = _compact(clickv, idxv, lambda v: v == clsv, t)

            sv[...] = jnp.zeros((16,), jnp.int32) + cnt
            pltpu.sync_copy(sv, cnt_h.at[bb])

            nch = (cnt + 127) // 128

            def gat(j, _):
                for k in range(8):
                    idxch[pl.ds(k * 16, 16)] = idxv[
                        pl.ds(j * 128 + k * 16, 16)]
                pltpu.async_copy(etr_h.at[bb].at[idxch], rows, sem).wait()
                pltpu.sync_copy(rows, bg_h.at[bb].at[pl.ds(j * 128, 128)])
                return 0
            lax.fori_loop(0, nch, gat, 0)

    return sck(sel, click, etr, cls_arr)


def _dotT(a, bmat):
    return lax.dot_general(a, bmat, (((1,), (1,)), ((), ())),
                           preferred_element_type=jnp.float32,
                           precision=_PREC)


def _rowsum(rowvec, colvec):
    return lax.dot_general(rowvec, colvec, (((1,), (0,)), ((), ())),
                           preferred_element_type=jnp.float32,
                           precision=_PREC)[0, 0]


def _main_body(cnt_ref, fg_ref, lab_ref, bg_ref, loss_ref, val_ref,
               fn_ref, bn_ref):
    nfg, d = fn_ref.shape
    t = bn_ref.shape[0]
    c = lab_ref.shape[2]
    nft = nfg // FT

    cnt = cnt_ref[0, 0, 0]
    c_bg = cnt.astype(jnp.float32)
    labf = (lab_ref[0] == 1).astype(jnp.float32)
    rsel = (lax.broadcasted_iota(jnp.int32, (c, nfg), 1) // K
            == lax.broadcasted_iota(jnp.int32, (c, nfg), 0)
            ).astype(jnp.float32)
    wgt = lax.dot_general(labf, rsel, (((1,), (0,)), ((), ())),
                          preferred_element_type=jnp.float32,
                          precision=_PREC)
    c_fg = jnp.sum(wgt)
    nbt = lax.div(cnt + (FT - 1), FT)

    for i in range(nft):
        rows = fg_ref[0, i * FT:(i + 1) * FT, :]
        n2 = jnp.sum(rows * rows, axis=1, keepdims=True)
        fn_ref[i * FT:(i + 1) * FT, :] = rows / jnp.sqrt(
            jnp.maximum(n2, 1e-24))

    def nrm(j, _):
        rows = bg_ref[0, pl.ds(j * FT, FT), :]
        n2 = jnp.sum(rows * rows, axis=1, keepdims=True)
        bn_ref[pl.ds(j * FT, FT), :] = rows / jnp.sqrt(
            jnp.maximum(n2, 1e-24))
        return 0
    lax.fori_loop(0, nbt, nrm, 0)

    s1 = jnp.float32(0.0)
    s3 = jnp.float32(0.0)
    for r in range(nft):
        fr = fn_ref[r * FT:(r + 1) * FT, :]
        w_r = wgt[:, r * FT:(r + 1) * FT]
        amin = jnp.full((FT, 1), jnp.inf, jnp.float32)
        for q in range(nft):
            g = _dotT(fr, fn_ref[q * FT:(q + 1) * FT, :])
            wm = wgt[:, q * FT:(q + 1) * FT] > 0
            amin = jnp.minimum(amin, jnp.min(
                jnp.where(wm, g, jnp.inf), axis=1, keepdims=True))
        s1 = s1 + _rowsum(w_r, jnp.maximum(TH_SIMILAR_MIN - amin, 0.0))

        def fb(j, cmax):
            g = _dotT(fr, bn_ref[pl.ds(j * FT, FT), :])
            lanes = lax.broadcasted_iota(jnp.int32, (1, FT), 1) + j * FT
            bm = lanes < cnt
            return jnp.maximum(cmax, jnp.max(
                jnp.where(bm, g, -jnp.inf), axis=1, keepdims=True))
        cmax = lax.fori_loop(0, nbt, fb,
                             jnp.full((FT, 1), -jnp.inf, jnp.float32))
        s3 = s3 + _rowsum(w_r, jnp.maximum(cmax - TH_DIFFERENT_MAX, 0.0))

    def brow(rj, s24):
        s2, s4 = s24
        br = bn_ref[pl.ds(rj * FT, FT), :]
        rmask = (lax.broadcasted_iota(jnp.int32, (FT, 1), 0)
                 + rj * FT) < cnt
        dmax = jnp.full((FT, 1), -jnp.inf, jnp.float32)
        for q in range(nft):
            g = _dotT(br, fn_ref[q * FT:(q + 1) * FT, :])
            wm = wgt[:, q * FT:(q + 1) * FT] > 0
            dmax = jnp.maximum(dmax, jnp.max(
                jnp.where(wm, g, -jnp.inf), axis=1, keepdims=True))
        rel4 = jnp.where(rmask,
                         jnp.maximum(dmax - TH_DIFFERENT_MAX, 0.0), 0.0)
        s4 = s4 + jnp.sum(rel4)

        def bb(cj, bmin):
            g = _dotT(br, bn_ref[pl.ds(cj * FT, FT), :])
            lanes = lax.broadcasted_iota(jnp.int32, (1, FT), 1) + cj * FT
            bm = lanes < cnt
            return jnp.minimum(bmin, jnp.min(
                jnp.where(bm, g, jnp.inf), axis=1, keepdims=True))
        bmin = lax.fori_loop(0, nbt, bb,
                             jnp.full((FT, 1), jnp.inf, jnp.float32))
        rel2 = jnp.where(rmask,
                         jnp.maximum(TH_SIMILAR_MIN - bmin, 0.0), 0.0)
        s2 = s2 + jnp.sum(rel2)
        return s2, s4

    s2, s4 = lax.fori_loop(0, nbt, brow,
                           (jnp.float32(0.0), jnp.float32(0.0)))

    loss_b = (s1 + s3) / c_fg + (s2 + s4) / c_bg
    valid = cnt > 0
    loss_ref[...] = jnp.zeros((1, 8, 128), jnp.float32) + jnp.where(
        valid, loss_b, 0.0)
    val_ref[...] = jnp.zeros((1, 8, 128), jnp.float32) + jnp.where(
        valid, 1.0, 0.0)


def _main(fgbuf, label, bgbuf, counts):
    b, nfg, d = fgbuf.shape
    t = bgbuf.shape[1]
    c = label.shape[1]
    lab3 = label.reshape(b, 1, c)
    cnt3 = counts.reshape(b, 1, 16)
    return pl.pallas_call(
        _main_body,
        grid=(b,),
        in_specs=[
            pl.BlockSpec((1, 1, 16), lambda i: (i, 0, 0),
                         memory_space=pltpu.SMEM),
            pl.BlockSpec((1, nfg, d), lambda i: (i, 0, 0)),
            pl.BlockSpec((1, 1, c), lambda i: (i, 0, 0)),
            pl.BlockSpec((1, t, d), lambda i: (i, 0, 0)),
        ],
        out_specs=[
            pl.BlockSpec((1, 8, 128), lambda i: (i, 0, 0)),
            pl.BlockSpec((1, 8, 128), lambda i: (i, 0, 0)),
        ],
        out_shape=[
            jax.ShapeDtypeStruct((b, 8, 128), jnp.float32),
            jax.ShapeDtypeStruct((b, 8, 128), jnp.float32),
        ],
        scratch_shapes=[
            pltpu.VMEM((nfg, d), jnp.float32),
            pltpu.VMEM((t, d), jnp.float32),
        ],
    )(cnt3, fgbuf, lab3, bgbuf)


def kernel(embeddings, click_labels, cas_supp, label, cls_num):
    b, d, t = embeddings.shape
    c = cas_supp.shape[1]

    sel = _topk_sel(cas_supp)
    etr = jnp.swapaxes(embeddings, 1, 2)
    cls_arr = jnp.zeros((16,), jnp.int32) + jnp.asarray(cls_num, jnp.int32)

    fgbuf, bgbuf, counts = _sc_gather(sel, click_labels, etr, cls_arr)
    lossm, validf = _main(fgbuf, label, bgbuf, counts)
    return jnp.sum(lossm[:, 0, 0]) / jnp.sum(validf[:, 0, 0])
